# baseline (speedup 1.0000x reference)
import functools

import jax
import jax.numpy as jnp
from jax import lax
from jax.experimental import pallas as pl
from jax.experimental.pallas import tpu as pltpu
from jax.experimental.pallas import tpu_sc as plsc

N = 10000
E = 320000
D = 128
DL = 64
NG = 16

NPAD = 10240
PT = NPAD // 16
SPAN = E // 16
CB = 2000
CPAD = 2048
NCH = SPAN // CB
BLK = 2048

_mesh = plsc.VectorSubcoreMesh(core_axis_name="c", subcore_axis_name="s")


def _splat(vec, jj):
    return jnp.broadcast_to(lax.slice(vec, (jj,), (jj + 1,)), (16,))


@functools.partial(
    pl.kernel,
    out_type=jax.ShapeDtypeStruct((2, NPAD), jnp.float32),
    mesh=_mesh,
    scratch_types=[
        pltpu.VMEM((CPAD,), jnp.int32),
        pltpu.VMEM((CPAD,), jnp.int32),
        pltpu.VMEM((CPAD,), jnp.float32),
        pltpu.VMEM((128,), jnp.int32),
        pltpu.VMEM((128,), jnp.float32),
        pltpu.VMEM((PT,), jnp.float32),
        pltpu.VMEM_SHARED((NPAD,), jnp.float32),
    ],
)
def _sc_degree(dst_hbm, et_hbm, ew_hbm, deg_hbm, dstv, etv, ewv, si, wv,
               onesb, acc):
    c = lax.axis_index("c")
    s = lax.axis_index("s")
    for j in range(PT // 16):
        onesb[pl.ds(j * 16, 16)] = jnp.ones((16,), jnp.float32)
    pltpu.sync_copy(onesb, acc.at[pl.ds(s * PT, PT)])
    plsc.subcore_barrier()

    def chunk(k, _):
        base = s * SPAN + k * CB
        pltpu.sync_copy(dst_hbm.at[pl.ds(base, CB)], dstv.at[pl.ds(0, CB)])
        pltpu.sync_copy(et_hbm.at[pl.ds(base, CB)], etv.at[pl.ds(0, CB)])
        pltpu.sync_copy(ew_hbm.at[pl.ds(base, CB)], ewv.at[pl.ds(0, CB)])
        for t in range((CPAD - CB) // 16):
            etv[pl.ds(CB + t * 16, 16)] = jnp.full((16,), 2, jnp.int32)

        def fire(b, _):
            for v in range(8):
                off = b * 128 + v * 16
                m = etv[pl.ds(off, 16)] == c
                si[pl.ds(v * 16, 16)] = jnp.where(m, dstv[pl.ds(off, 16)], 0)
                wv[pl.ds(v * 16, 16)] = jnp.where(m, ewv[pl.ds(off, 16)], 0.0)
            pltpu.sync_copy(wv, acc.at[si], add=True)
            return 0

        lax.fori_loop(0, CPAD // 128, fire, 0)
        return 0

    lax.fori_loop(0, NCH, chunk, 0)
    plsc.subcore_barrier()
    pltpu.sync_copy(acc.at[pl.ds(s * PT, PT)], deg_hbm.at[c, pl.ds(s * PT, PT)])


@functools.partial(
    pl.kernel,
    out_type=jax.ShapeDtypeStruct((2 * NPAD, D), jnp.float32),
    mesh=_mesh,
    scratch_types=[
        pltpu.VMEM((CPAD,), jnp.int32),
        pltpu.VMEM((CPAD,), jnp.int32),
        pltpu.VMEM((CPAD,), jnp.int32),
        pltpu.VMEM((CPAD,), jnp.float32),
        [pltpu.VMEM((128,), jnp.int32) for _ in range(2)],
        [pltpu.VMEM((128,), jnp.int32) for _ in range(2)],
        [pltpu.VMEM((128,), jnp.float32) for _ in range(2)],
        [pltpu.VMEM((128, D), jnp.float32) for _ in range(2)],
        pltpu.VMEM_SHARED((NPAD, D), jnp.float32),
        pltpu.SemaphoreType.DMA,
        pltpu.SemaphoreType.DMA,
    ],
)
def _sc_spmm(g_hbm, src_hbm, dst_hbm, et_hbm, ew_hbm, s_hbm,
             srcv, dstv, etv, ewv, gi, si, wv, rows, acc, sem_g, sem_s):
    c = lax.axis_index("c")
    s = lax.axis_index("s")

    def zrow(i, _):
        for cb in range(8):
            rows[0][i, pl.ds(cb * 16, 16)] = jnp.zeros((16,), jnp.float32)
        return 0

    lax.fori_loop(0, 128, zrow, 0)
    for q in range(PT // 128):
        pltpu.sync_copy(rows[0], acc.at[pl.ds(s * PT + q * 128, 128), :])
    plsc.subcore_barrier()

    def chunk(k, _):
        base = s * SPAN + k * CB
        pltpu.sync_copy(src_hbm.at[pl.ds(base, CB)], srcv.at[pl.ds(0, CB)])
        pltpu.sync_copy(dst_hbm.at[pl.ds(base, CB)], dstv.at[pl.ds(0, CB)])
        pltpu.sync_copy(et_hbm.at[pl.ds(base, CB)], etv.at[pl.ds(0, CB)])
        pltpu.sync_copy(ew_hbm.at[pl.ds(base, CB)], ewv.at[pl.ds(0, CB)])
        for t in range((CPAD - CB) // 16):
            etv[pl.ds(CB + t * 16, 16)] = jnp.full((16,), 2, jnp.int32)

        def group(grp, _):
            for b in range(2):
                for v in range(8):
                    off = grp * 256 + b * 128 + v * 16
                    m = etv[pl.ds(off, 16)] == c
                    gi[b][pl.ds(v * 16, 16)] = (
                        jnp.where(m, srcv[pl.ds(off, 16)], 0) + c * NPAD)
                    si[b][pl.ds(v * 16, 16)] = jnp.where(
                        m, dstv[pl.ds(off, 16)], 0)
                    wv[b][pl.ds(v * 16, 16)] = jnp.where(
                        m, ewv[pl.ds(off, 16)], 0.0)
            descs = [pltpu.async_copy(g_hbm.at[gi[b]], rows[b], sem_g)
                     for b in range(2)]
            for d in descs:
                d.wait()
            for b in range(2):
                def srow(rb, _, b=b):
                    ewb = wv[b][pl.ds(rb * 16, 16)]
                    for jj in range(16):
                        es = _splat(ewb, jj)
                        r = rb * 16 + jj
                        for cb in range(8):
                            rows[b][r, pl.ds(cb * 16, 16)] = (
                                rows[b][r, pl.ds(cb * 16, 16)] * es)
                    return 0

                lax.fori_loop(0, 8, srow, 0)
            descs = [pltpu.async_copy(rows[b], acc.at[si[b]], sem_s,
                                      add=True)
                     for b in range(2)]
            for d in descs:
                d.wait()
            return 0

        lax.fori_loop(0, CPAD // 256, group, 0)
        return 0

    lax.fori_loop(0, NCH, chunk, 0)
    plsc.subcore_barrier()
    pltpu.sync_copy(acc.at[pl.ds(s * PT, PT), :],
                    s_hbm.at[pl.ds(c * NPAD + s * PT, PT), :])


def _prep_body(degp_ref, degn_ref, x_ref, g_ref, disp_ref, disn_ref):
    i = pl.program_id(0)
    rel = i // (NPAD // BLK)
    dp = lax.rsqrt(degp_ref[...])
    dn = lax.rsqrt(degn_ref[...])
    disp_ref[...] = dp
    disn_ref[...] = dn
    g_ref[...] = x_ref[...] * jnp.where(rel == 0, dp, dn)


def _tc_prep(deg_p, deg_n, x):
    nb = NPAD // BLK
    return pl.pallas_call(
        _prep_body,
        grid=(2 * nb,),
        in_specs=[
            pl.BlockSpec((BLK, 1), lambda i: (i % nb, 0)),
            pl.BlockSpec((BLK, 1), lambda i: (i % nb, 0)),
            pl.BlockSpec((BLK, D), lambda i: (i % nb, 0)),
        ],
        out_specs=[
            pl.BlockSpec((BLK, D), lambda i: (i, 0)),
            pl.BlockSpec((BLK, 1), lambda i: (i % nb, 0)),
            pl.BlockSpec((BLK, 1), lambda i: (i % nb, 0)),
        ],
        out_shape=[
            jax.ShapeDtypeStruct((2 * NPAD, D), jnp.float32),
            jax.ShapeDtypeStruct((NPAD, 1), jnp.float32),
            jax.ShapeDtypeStruct((NPAD, 1), jnp.float32),
        ],
    )(deg_p, deg_n, x)


def _combine_body(sp_ref, sn_ref, h_ref, dp_ref, dn_ref,
                  wp_ref, wn_ref, bp_ref, bn_ref, hn_ref, g_ref):
    i = pl.program_id(0)
    rel = i // (NPAD // BLK)
    dp = dp_ref[...]
    dn = dn_ref[...]
    h = h_ref[...]
    up = dp * sp_ref[...] + (dp * dp) * h
    un = dn * sn_ref[...] + (dn * dn) * h
    hn = jnp.dot(up, wp_ref[...], preferred_element_type=jnp.float32)
    hn += jnp.dot(un, wn_ref[...], preferred_element_type=jnp.float32)
    hn = jnp.maximum(hn + bp_ref[...] + bn_ref[...], 0.0)
    hn_ref[...] = hn
    g_ref[...] = hn * jnp.where(rel == 0, dp, dn)


def _tc_combine(S_p, S_n, h, dis_p, dis_n, Wp, Wn, bp, bn):
    nb = NPAD // BLK
    return pl.pallas_call(
        _combine_body,
        grid=(2 * nb,),
        in_specs=[
            pl.BlockSpec((BLK, D), lambda i: (i % nb, 0)),
            pl.BlockSpec((BLK, D), lambda i: (i % nb, 0)),
            pl.BlockSpec((BLK, D), lambda i: (i % nb, 0)),
            pl.BlockSpec((BLK, 1), lambda i: (i % nb, 0)),
            pl.BlockSpec((BLK, 1), lambda i: (i % nb, 0)),
            pl.BlockSpec((D, D), lambda i: (0, 0)),
            pl.BlockSpec((D, D), lambda i: (0, 0)),
            pl.BlockSpec((1, D), lambda i: (0, 0)),
            pl.BlockSpec((1, D), lambda i: (0, 0)),
        ],
        out_specs=[
            pl.BlockSpec((BLK, D), lambda i: (i % nb, 0)),
            pl.BlockSpec((BLK, D), lambda i: (i, 0)),
        ],
        out_shape=[
            jax.ShapeDtypeStruct((N, D), jnp.float32),
            jax.ShapeDtypeStruct((2 * NPAD, D), jnp.float32),
        ],
    )(S_p, S_n, h, dis_p, dis_n, Wp, Wn, bp, bn)


def _final_body(sp_ref, sn_ref, h_ref, dp_ref, dn_ref, wp_ref, wn_ref,
                bp_ref, bn_ref, batch_ref, zn_ref, zg_ref, sums, cnts):
    i = pl.program_id(0)
    nb = NPAD // BLK
    dp = dp_ref[...]
    dn = dn_ref[...]
    h = h_ref[...]
    up = dp * sp_ref[...] + (dp * dp) * h
    un = dn * sn_ref[...] + (dn * dn) * h
    z = jnp.dot(up, wp_ref[...], preferred_element_type=jnp.float32)
    z += jnp.dot(un, wn_ref[...], preferred_element_type=jnp.float32)
    z = z + bp_ref[...] + bn_ref[...]
    zn_ref[...] = z

    b = batch_ref[...]
    valid = b < float(NG)
    zm = jnp.where(valid, z, 0.0)
    gids = lax.broadcasted_iota(jnp.int32, (1, NG), 1).astype(jnp.float32)
    oh = (b == gids).astype(jnp.float32)

    @pl.when(i == 0)
    def _():
        sums[...] = jnp.zeros_like(sums)
        cnts[...] = jnp.zeros_like(cnts)

    sums[...] += lax.dot_general(oh, zm, (((0,), (0,)), ((), ())),
                                 preferred_element_type=jnp.float32)
    cnt = jnp.sum(oh, axis=0)
    cnts[...] += jnp.broadcast_to(cnt[:, None], (NG, DL))

    @pl.when(i == nb - 1)
    def _():
        zg_ref[...] = sums[...] / jnp.maximum(cnts[...], 1.0)


def _tc_final(S_p, S_n, h, dis_p, dis_n, Wp, Wn, bp, bn, batchf):
    nb = NPAD // BLK
    return pl.pallas_call(
        _final_body,
        grid=(nb,),
        in_specs=[
            pl.BlockSpec((BLK, D), lambda i: (i, 0)),
            pl.BlockSpec((BLK, D), lambda i: (i, 0)),
            pl.BlockSpec((BLK, D), lambda i: (i, 0)),
            pl.BlockSpec((BLK, 1), lambda i: (i, 0)),
            pl.BlockSpec((BLK, 1), lambda i: (i, 0)),
            pl.BlockSpec((D, DL), lambda i: (0, 0)),
            pl.BlockSpec((D, DL), lambda i: (0, 0)),
            pl.BlockSpec((1, DL), lambda i: (0, 0)),
            pl.BlockSpec((1, DL), lambda i: (0, 0)),
            pl.BlockSpec((BLK, 1), lambda i: (i, 0)),
        ],
        out_specs=[
            pl.BlockSpec((BLK, DL), lambda i: (i, 0)),
            pl.BlockSpec((NG, DL), lambda i: (0, 0)),
        ],
        out_shape=[
            jax.ShapeDtypeStruct((N, DL), jnp.float32),
            jax.ShapeDtypeStruct((NG, DL), jnp.float32),
        ],
        scratch_shapes=[
            pltpu.VMEM((NG, DL), jnp.float32),
            pltpu.VMEM((NG, DL), jnp.float32),
        ],
    )(S_p, S_n, h, dis_p, dis_n, Wp, Wn, bp, bn, batchf)


def kernel(x, edge_index, edge_type, edge_attr, batch,
           W1p, b1p, W1n, b1n, W2p, b2p, W2n, b2n, W3p, b3p, W3n, b3n):
    src = edge_index[0]
    dst = edge_index[1]

    deg = _sc_degree(dst, edge_type, edge_attr)
    deg_p = deg[0].reshape(NPAD, 1)
    deg_n = deg[1].reshape(NPAD, 1)

    g, dis_p, dis_n = _tc_prep(deg_p, deg_n, x)

    S = _sc_spmm(g, src, dst, edge_type, edge_attr)
    h1, g = _tc_combine(S[:NPAD], S[NPAD:], x, dis_p, dis_n,
                        W1p, W1n, b1p.reshape(1, D), b1n.reshape(1, D))

    S = _sc_spmm(g, src, dst, edge_type, edge_attr)
    h2, g = _tc_combine(S[:NPAD], S[NPAD:], h1, dis_p, dis_n,
                        W2p, W2n, b2p.reshape(1, D), b2n.reshape(1, D))

    S = _sc_spmm(g, src, dst, edge_type, edge_attr)
    batchf = jnp.concatenate(
        [batch.astype(jnp.float32),
         jnp.full((NPAD - N,), 99.0, jnp.float32)]).reshape(NPAD, 1)
    z_node, z_graph = _tc_final(S[:NPAD], S[NPAD:], h2, dis_p, dis_n,
                                W3p, W3n, b3p.reshape(1, DL),
                                b3n.reshape(1, DL), batchf)
    return (z_node, z_graph)


# speedup vs baseline: 2.7428x; 2.7428x over previous
import functools

import jax
import jax.numpy as jnp
from jax import lax
from jax.experimental import pallas as pl
from jax.experimental.pallas import tpu as pltpu
from jax.experimental.pallas import tpu_sc as plsc

N = 10000
E = 320000
D = 128
DL = 64
NG = 16

NPAD = 10240
PT = NPAD // 16
SPAN = E // 16
CB = 2000
CPAD = 2048
NCH = SPAN // CB
BLK = 2048

_mesh = plsc.VectorSubcoreMesh(core_axis_name="c", subcore_axis_name="s")


def _lane_gather(vec, idx):
    return lax.gather(
        vec, idx[:, None],
        dimension_numbers=lax.GatherDimensionNumbers(
            offset_dims=(), collapsed_slice_dims=(0,), start_index_map=(0,)),
        slice_sizes=(1,),
        mode=lax.GatherScatterMode.PROMISE_IN_BOUNDS)


def _splat(vec, jj):
    return jnp.broadcast_to(lax.slice(vec, (jj,), (jj + 1,)), (16,))


@functools.partial(
    pl.kernel,
    out_type=jax.ShapeDtypeStruct((2, NPAD), jnp.float32),
    mesh=_mesh,
    scratch_types=[
        pltpu.VMEM((CPAD,), jnp.int32),
        pltpu.VMEM((CPAD,), jnp.int32),
        pltpu.VMEM((CPAD,), jnp.float32),
        pltpu.VMEM((128,), jnp.int32),
        pltpu.VMEM((128,), jnp.float32),
        pltpu.VMEM((PT,), jnp.float32),
        pltpu.VMEM_SHARED((NPAD,), jnp.float32),
    ],
)
def _sc_degree(dst_hbm, et_hbm, ew_hbm, deg_hbm, dstv, etv, ewv, si, wv,
               onesb, acc):
    c = lax.axis_index("c")
    s = lax.axis_index("s")
    for j in range(PT // 16):
        onesb[pl.ds(j * 16, 16)] = jnp.ones((16,), jnp.float32)
    pltpu.sync_copy(onesb, acc.at[pl.ds(s * PT, PT)])
    plsc.subcore_barrier()

    def chunk(k, _):
        base = s * SPAN + k * CB
        pltpu.sync_copy(dst_hbm.at[pl.ds(base, CB)], dstv.at[pl.ds(0, CB)])
        pltpu.sync_copy(et_hbm.at[pl.ds(base, CB)], etv.at[pl.ds(0, CB)])
        pltpu.sync_copy(ew_hbm.at[pl.ds(base, CB)], ewv.at[pl.ds(0, CB)])
        for t in range((CPAD - CB) // 16):
            etv[pl.ds(CB + t * 16, 16)] = jnp.full((16,), 2, jnp.int32)

        def fire(b, _):
            for v in range(8):
                off = b * 128 + v * 16
                m = etv[pl.ds(off, 16)] == c
                si[pl.ds(v * 16, 16)] = jnp.where(m, dstv[pl.ds(off, 16)], 0)
                wv[pl.ds(v * 16, 16)] = jnp.where(m, ewv[pl.ds(off, 16)], 0.0)
            pltpu.sync_copy(wv, acc.at[si], add=True)
            return 0

        lax.fori_loop(0, CPAD // 128, fire, 0)
        return 0

    lax.fori_loop(0, NCH, chunk, 0)
    plsc.subcore_barrier()
    pltpu.sync_copy(acc.at[pl.ds(s * PT, PT)], deg_hbm.at[c, pl.ds(s * PT, PT)])


PASSES = 4
PR = NPAD // PASSES
PRT = PR // 16


@functools.partial(
    pl.kernel,
    out_type=jax.ShapeDtypeStruct((2 * NPAD, D), jnp.float32),
    mesh=_mesh,
    scratch_types=[
        pltpu.VMEM((CPAD,), jnp.int32),
        pltpu.VMEM((CPAD,), jnp.int32),
        pltpu.VMEM((CPAD,), jnp.int32),
        pltpu.VMEM((CPAD,), jnp.float32),
        pltpu.VMEM((128,), jnp.int32),
        pltpu.VMEM((128,), jnp.int32),
        pltpu.VMEM((128,), jnp.float32),
        pltpu.VMEM((128, D), jnp.float32),
        pltpu.VMEM_SHARED((NPAD, D), jnp.float32),
        pltpu.VMEM_SHARED((PR, D), jnp.float32),
    ],
)
def _sc_spmm(g_hbm, src_hbm, dst_hbm, et_hbm, ew_hbm, s_hbm,
             srcv, dstv, etv, ewv, gi, si, wv, rows, g_sh, acc):
    c = lax.axis_index("c")
    s = lax.axis_index("s")

    pltpu.sync_copy(g_hbm.at[pl.ds(c * NPAD + s * PT, PT), :],
                    g_sh.at[pl.ds(s * PT, PT), :])

    def zrow(i, _):
        for cb in range(8):
            rows[i, pl.ds(cb * 16, 16)] = jnp.zeros((16,), jnp.float32)
        return 0

    lax.fori_loop(0, 128, zrow, 0)

    def apass(p, _):
        lo = p * PR
        lax.fori_loop(0, 128, zrow, 0)
        nz = PRT // 128
        for q in range(nz):
            pltpu.sync_copy(rows, acc.at[pl.ds(s * PRT + q * 128, 128), :])
        if PRT % 128:
            pltpu.sync_copy(
                rows.at[pl.ds(0, PRT % 128), :],
                acc.at[pl.ds(s * PRT + nz * 128, PRT % 128), :])
        plsc.subcore_barrier()

        def chunk(k, _):
            base = s * SPAN + k * CB
            pltpu.sync_copy(src_hbm.at[pl.ds(base, CB)], srcv.at[pl.ds(0, CB)])
            pltpu.sync_copy(dst_hbm.at[pl.ds(base, CB)], dstv.at[pl.ds(0, CB)])
            pltpu.sync_copy(et_hbm.at[pl.ds(base, CB)], etv.at[pl.ds(0, CB)])
            pltpu.sync_copy(ew_hbm.at[pl.ds(base, CB)], ewv.at[pl.ds(0, CB)])
            for t in range((CPAD - CB) // 16):
                etv[pl.ds(CB + t * 16, 16)] = jnp.full((16,), 2, jnp.int32)

            def fire(b, _):
                for v in range(8):
                    off = b * 128 + v * 16
                    dl = dstv[pl.ds(off, 16)] - lo
                    mi = (jnp.where(etv[pl.ds(off, 16)] == c, 1, 0)
                          * jnp.where(dl >= 0, 1, 0)
                          * jnp.where(dl < PR, 1, 0))
                    ok = mi == 1
                    gi[pl.ds(v * 16, 16)] = jnp.where(
                        ok, srcv[pl.ds(off, 16)], 0)
                    si[pl.ds(v * 16, 16)] = jnp.where(ok, dl, 0)
                    wv[pl.ds(v * 16, 16)] = jnp.where(
                        ok, ewv[pl.ds(off, 16)], 0.0)
                pltpu.sync_copy(g_sh.at[gi], rows)

                def srow(rb, _):
                    ewb = wv[pl.ds(rb * 16, 16)]
                    for jj in range(16):
                        es = _splat(ewb, jj)
                        r = rb * 16 + jj
                        for cb in range(8):
                            rows[r, pl.ds(cb * 16, 16)] = (
                                rows[r, pl.ds(cb * 16, 16)] * es)
                    return 0

                lax.fori_loop(0, 8, srow, 0)
                pltpu.sync_copy(rows, acc.at[si], add=True)
                return 0

            lax.fori_loop(0, CPAD // 128, fire, 0)
            return 0

        lax.fori_loop(0, NCH, chunk, 0)
        plsc.subcore_barrier()
        pltpu.sync_copy(
            acc.at[pl.ds(s * PRT, PRT), :],
            s_hbm.at[pl.ds(c * NPAD + p * PR + s * PRT, PRT), :])
        return 0

    lax.fori_loop(0, PASSES, apass, 0)


def _prep_body(degp_ref, degn_ref, x_ref, g_ref, disp_ref, disn_ref):
    i = pl.program_id(0)
    rel = i // (NPAD // BLK)
    dp = lax.rsqrt(degp_ref[...])
    dn = lax.rsqrt(degn_ref[...])
    disp_ref[...] = dp
    disn_ref[...] = dn
    g_ref[...] = x_ref[...] * jnp.where(rel == 0, dp, dn)


def _tc_prep(deg_p, deg_n, x):
    nb = NPAD // BLK
    return pl.pallas_call(
        _prep_body,
        grid=(2 * nb,),
        in_specs=[
            pl.BlockSpec((BLK, 1), lambda i: (i % nb, 0)),
            pl.BlockSpec((BLK, 1), lambda i: (i % nb, 0)),
            pl.BlockSpec((BLK, D), lambda i: (i % nb, 0)),
        ],
        out_specs=[
            pl.BlockSpec((BLK, D), lambda i: (i, 0)),
            pl.BlockSpec((BLK, 1), lambda i: (i % nb, 0)),
            pl.BlockSpec((BLK, 1), lambda i: (i % nb, 0)),
        ],
        out_shape=[
            jax.ShapeDtypeStruct((2 * NPAD, D), jnp.float32),
            jax.ShapeDtypeStruct((NPAD, 1), jnp.float32),
            jax.ShapeDtypeStruct((NPAD, 1), jnp.float32),
        ],
    )(deg_p, deg_n, x)


def _combine_body(sp_ref, sn_ref, h_ref, dp_ref, dn_ref,
                  wp_ref, wn_ref, bp_ref, bn_ref, hn_ref, g_ref):
    i = pl.program_id(0)
    rel = i // (NPAD // BLK)
    dp = dp_ref[...]
    dn = dn_ref[...]
    h = h_ref[...]
    up = dp * sp_ref[...] + (dp * dp) * h
    un = dn * sn_ref[...] + (dn * dn) * h
    hn = jnp.dot(up, wp_ref[...], preferred_element_type=jnp.float32)
    hn += jnp.dot(un, wn_ref[...], preferred_element_type=jnp.float32)
    hn = jnp.maximum(hn + bp_ref[...] + bn_ref[...], 0.0)
    hn_ref[...] = hn
    g_ref[...] = hn * jnp.where(rel == 0, dp, dn)


def _tc_combine(S_p, S_n, h, dis_p, dis_n, Wp, Wn, bp, bn):
    nb = NPAD // BLK
    return pl.pallas_call(
        _combine_body,
        grid=(2 * nb,),
        in_specs=[
            pl.BlockSpec((BLK, D), lambda i: (i % nb, 0)),
            pl.BlockSpec((BLK, D), lambda i: (i % nb, 0)),
            pl.BlockSpec((BLK, D), lambda i: (i % nb, 0)),
            pl.BlockSpec((BLK, 1), lambda i: (i % nb, 0)),
            pl.BlockSpec((BLK, 1), lambda i: (i % nb, 0)),
            pl.BlockSpec((D, D), lambda i: (0, 0)),
            pl.BlockSpec((D, D), lambda i: (0, 0)),
            pl.BlockSpec((1, D), lambda i: (0, 0)),
            pl.BlockSpec((1, D), lambda i: (0, 0)),
        ],
        out_specs=[
            pl.BlockSpec((BLK, D), lambda i: (i % nb, 0)),
            pl.BlockSpec((BLK, D), lambda i: (i, 0)),
        ],
        out_shape=[
            jax.ShapeDtypeStruct((N, D), jnp.float32),
            jax.ShapeDtypeStruct((2 * NPAD, D), jnp.float32),
        ],
    )(S_p, S_n, h, dis_p, dis_n, Wp, Wn, bp, bn)


def _final_body(sp_ref, sn_ref, h_ref, dp_ref, dn_ref, wp_ref, wn_ref,
                bp_ref, bn_ref, batch_ref, zn_ref, zg_ref, sums, cnts):
    i = pl.program_id(0)
    nb = NPAD // BLK
    dp = dp_ref[...]
    dn = dn_ref[...]
    h = h_ref[...]
    up = dp * sp_ref[...] + (dp * dp) * h
    un = dn * sn_ref[...] + (dn * dn) * h
    z = jnp.dot(up, wp_ref[...], preferred_element_type=jnp.float32)
    z += jnp.dot(un, wn_ref[...], preferred_element_type=jnp.float32)
    z = z + bp_ref[...] + bn_ref[...]
    zn_ref[...] = z

    b = batch_ref[...]
    valid = b < float(NG)
    zm = jnp.where(valid, z, 0.0)
    gids = lax.broadcasted_iota(jnp.int32, (1, NG), 1).astype(jnp.float32)
    oh = (b == gids).astype(jnp.float32)

    @pl.when(i == 0)
    def _():
        sums[...] = jnp.zeros_like(sums)
        cnts[...] = jnp.zeros_like(cnts)

    sums[...] += lax.dot_general(oh, zm, (((0,), (0,)), ((), ())),
                                 preferred_element_type=jnp.float32)
    cnt = jnp.sum(oh, axis=0)
    cnts[...] += jnp.broadcast_to(cnt[:, None], (NG, DL))

    @pl.when(i == nb - 1)
    def _():
        zg_ref[...] = sums[...] / jnp.maximum(cnts[...], 1.0)


def _tc_final(S_p, S_n, h, dis_p, dis_n, Wp, Wn, bp, bn, batchf):
    nb = NPAD // BLK
    return pl.pallas_call(
        _final_body,
        grid=(nb,),
        in_specs=[
            pl.BlockSpec((BLK, D), lambda i: (i, 0)),
            pl.BlockSpec((BLK, D), lambda i: (i, 0)),
            pl.BlockSpec((BLK, D), lambda i: (i, 0)),
            pl.BlockSpec((BLK, 1), lambda i: (i, 0)),
            pl.BlockSpec((BLK, 1), lambda i: (i, 0)),
            pl.BlockSpec((D, DL), lambda i: (0, 0)),
            pl.BlockSpec((D, DL), lambda i: (0, 0)),
            pl.BlockSpec((1, DL), lambda i: (0, 0)),
            pl.BlockSpec((1, DL), lambda i: (0, 0)),
            pl.BlockSpec((BLK, 1), lambda i: (i, 0)),
        ],
        out_specs=[
            pl.BlockSpec((BLK, DL), lambda i: (i, 0)),
            pl.BlockSpec((NG, DL), lambda i: (0, 0)),
        ],
        out_shape=[
            jax.ShapeDtypeStruct((N, DL), jnp.float32),
            jax.ShapeDtypeStruct((NG, DL), jnp.float32),
        ],
        scratch_shapes=[
            pltpu.VMEM((NG, DL), jnp.float32),
            pltpu.VMEM((NG, DL), jnp.float32),
        ],
    )(S_p, S_n, h, dis_p, dis_n, Wp, Wn, bp, bn, batchf)


def kernel(x, edge_index, edge_type, edge_attr, batch,
           W1p, b1p, W1n, b1n, W2p, b2p, W2n, b2n, W3p, b3p, W3n, b3n):
    src = edge_index[0]
    dst = edge_index[1]

    deg = _sc_degree(dst, edge_type, edge_attr)
    deg_p = deg[0].reshape(NPAD, 1)
    deg_n = deg[1].reshape(NPAD, 1)

    g, dis_p, dis_n = _tc_prep(deg_p, deg_n, x)

    S = _sc_spmm(g, src, dst, edge_type, edge_attr)
    h1, g = _tc_combine(S[:NPAD], S[NPAD:], x, dis_p, dis_n,
                        W1p, W1n, b1p.reshape(1, D), b1n.reshape(1, D))

    S = _sc_spmm(g, src, dst, edge_type, edge_attr)
    h2, g = _tc_combine(S[:NPAD], S[NPAD:], h1, dis_p, dis_n,
                        W2p, W2n, b2p.reshape(1, D), b2n.reshape(1, D))

    S = _sc_spmm(g, src, dst, edge_type, edge_attr)
    batchf = jnp.concatenate(
        [batch.astype(jnp.float32),
         jnp.full((NPAD - N,), 99.0, jnp.float32)]).reshape(NPAD, 1)
    z_node, z_graph = _tc_final(S[:NPAD], S[NPAD:], h2, dis_p, dis_n,
                                W3p, W3n, b3p.reshape(1, DL),
                                b3n.reshape(1, DL), batchf)
    return (z_node, z_graph)


# speedup vs baseline: 2.7451x; 1.0008x over previous
import functools

import jax
import jax.numpy as jnp
from jax import lax
from jax.experimental import pallas as pl
from jax.experimental.pallas import tpu as pltpu
from jax.experimental.pallas import tpu_sc as plsc

N = 10000
E = 320000
D = 128
DL = 64
NG = 16

NPAD = 10240
PT = NPAD // 16
SPAN = E // 16
CB = 2000
CPAD = 2048
NCH = SPAN // CB
BLK = 2048

_mesh = plsc.VectorSubcoreMesh(core_axis_name="c", subcore_axis_name="s")


def _lane_gather(vec, idx):
    return lax.gather(
        vec, idx[:, None],
        dimension_numbers=lax.GatherDimensionNumbers(
            offset_dims=(), collapsed_slice_dims=(0,), start_index_map=(0,)),
        slice_sizes=(1,),
        mode=lax.GatherScatterMode.PROMISE_IN_BOUNDS)


def _splat(vec, jj):
    return jnp.broadcast_to(lax.slice(vec, (jj,), (jj + 1,)), (16,))


@functools.partial(
    pl.kernel,
    out_type=jax.ShapeDtypeStruct((2, NPAD), jnp.float32),
    mesh=_mesh,
    scratch_types=[
        pltpu.VMEM((CPAD,), jnp.int32),
        pltpu.VMEM((CPAD,), jnp.int32),
        pltpu.VMEM((CPAD,), jnp.float32),
        pltpu.VMEM((128,), jnp.int32),
        pltpu.VMEM((128,), jnp.float32),
        pltpu.VMEM((PT,), jnp.float32),
        pltpu.VMEM_SHARED((NPAD,), jnp.float32),
    ],
)
def _sc_degree(dst_hbm, et_hbm, ew_hbm, deg_hbm, dstv, etv, ewv, si, wv,
               onesb, acc):
    c = lax.axis_index("c")
    s = lax.axis_index("s")
    for j in range(PT // 16):
        onesb[pl.ds(j * 16, 16)] = jnp.ones((16,), jnp.float32)
    pltpu.sync_copy(onesb, acc.at[pl.ds(s * PT, PT)])
    plsc.subcore_barrier()

    def chunk(k, _):
        base = s * SPAN + k * CB
        pltpu.sync_copy(dst_hbm.at[pl.ds(base, CB)], dstv.at[pl.ds(0, CB)])
        pltpu.sync_copy(et_hbm.at[pl.ds(base, CB)], etv.at[pl.ds(0, CB)])
        pltpu.sync_copy(ew_hbm.at[pl.ds(base, CB)], ewv.at[pl.ds(0, CB)])
        for t in range((CPAD - CB) // 16):
            etv[pl.ds(CB + t * 16, 16)] = jnp.full((16,), 2, jnp.int32)

        def fire(b, _):
            for v in range(8):
                off = b * 128 + v * 16
                m = etv[pl.ds(off, 16)] == c
                si[pl.ds(v * 16, 16)] = jnp.where(m, dstv[pl.ds(off, 16)], 0)
                wv[pl.ds(v * 16, 16)] = jnp.where(m, ewv[pl.ds(off, 16)], 0.0)
            pltpu.sync_copy(wv, acc.at[si], add=True)
            return 0

        lax.fori_loop(0, CPAD // 128, fire, 0)
        return 0

    lax.fori_loop(0, NCH, chunk, 0)
    plsc.subcore_barrier()
    pltpu.sync_copy(acc.at[pl.ds(s * PT, PT)], deg_hbm.at[c, pl.ds(s * PT, PT)])


PASSES = 4
PR = NPAD // PASSES
PRT = PR // 16


@functools.partial(
    pl.kernel,
    out_type=jax.ShapeDtypeStruct((2 * NPAD, D), jnp.float32),
    mesh=_mesh,
    scratch_types=[
        pltpu.VMEM((CPAD,), jnp.int32),
        pltpu.VMEM((CPAD,), jnp.int32),
        pltpu.VMEM((CPAD,), jnp.int32),
        pltpu.VMEM((CPAD,), jnp.float32),
        pltpu.VMEM((128,), jnp.int32),
        pltpu.VMEM((128,), jnp.int32),
        pltpu.VMEM((128,), jnp.float32),
        pltpu.VMEM((128, D), jnp.float32),
        pltpu.VMEM_SHARED((NPAD, D), jnp.float32),
        pltpu.VMEM_SHARED((PR, D), jnp.float32),
    ],
)
def _sc_spmm(g_hbm, src_hbm, dst_hbm, et_hbm, ew_hbm, s_hbm,
             srcv, dstv, etv, ewv, gi, si, wv, rows, g_sh, acc):
    c = lax.axis_index("c")
    s = lax.axis_index("s")

    pltpu.sync_copy(g_hbm.at[pl.ds(c * NPAD + s * PT, PT), :],
                    g_sh.at[pl.ds(s * PT, PT), :])

    def zrow(i, _):
        for cb in range(8):
            rows[i, pl.ds(cb * 16, 16)] = jnp.zeros((16,), jnp.float32)
        return 0

    def apass(p, _):
        lo = p * PR
        lax.fori_loop(0, 128, zrow, 0)
        nz = PRT // 128
        for q in range(nz):
            pltpu.sync_copy(rows, acc.at[pl.ds(s * PRT + q * 128, 128), :])
        if PRT % 128:
            pltpu.sync_copy(
                rows.at[pl.ds(0, PRT % 128), :],
                acc.at[pl.ds(s * PRT + nz * 128, PRT % 128), :])
        plsc.subcore_barrier()

        def chunk(k, _):
            base = s * SPAN + k * CB
            pltpu.sync_copy(src_hbm.at[pl.ds(base, CB)], srcv.at[pl.ds(0, CB)])
            pltpu.sync_copy(dst_hbm.at[pl.ds(base, CB)], dstv.at[pl.ds(0, CB)])
            pltpu.sync_copy(et_hbm.at[pl.ds(base, CB)], etv.at[pl.ds(0, CB)])
            pltpu.sync_copy(ew_hbm.at[pl.ds(base, CB)], ewv.at[pl.ds(0, CB)])
            for t in range((CPAD - CB) // 16):
                etv[pl.ds(CB + t * 16, 16)] = jnp.full((16,), 2, jnp.int32)

            def fire(b, _):
                for v in range(8):
                    off = b * 128 + v * 16
                    dl = dstv[pl.ds(off, 16)] - lo
                    mi = (jnp.where(etv[pl.ds(off, 16)] == c, 1, 0)
                          * jnp.where(dl >= 0, 1, 0)
                          * jnp.where(dl < PR, 1, 0))
                    ok = mi == 1
                    gi[pl.ds(v * 16, 16)] = jnp.where(
                        ok, srcv[pl.ds(off, 16)], 0)
                    si[pl.ds(v * 16, 16)] = jnp.where(ok, dl, 0)
                    wv[pl.ds(v * 16, 16)] = jnp.where(
                        ok, ewv[pl.ds(off, 16)], 0.0)
                pltpu.sync_copy(g_sh.at[gi], rows)

                def srow(rb, _):
                    ewb = wv[pl.ds(rb * 16, 16)]
                    for jj in range(16):
                        es = _splat(ewb, jj)
                        r = rb * 16 + jj
                        for cb in range(8):
                            rows[r, pl.ds(cb * 16, 16)] = (
                                rows[r, pl.ds(cb * 16, 16)] * es)
                    return 0

                lax.fori_loop(0, 8, srow, 0)
                pltpu.sync_copy(rows, acc.at[si], add=True)
                return 0

            lax.fori_loop(0, CPAD // 128, fire, 0)
            return 0

        lax.fori_loop(0, NCH, chunk, 0)
        plsc.subcore_barrier()
        pltpu.sync_copy(
            acc.at[pl.ds(s * PRT, PRT), :],
            s_hbm.at[pl.ds(c * NPAD + p * PR + s * PRT, PRT), :])
        return 0

    lax.fori_loop(0, PASSES, apass, 0)


def _prep_body(degp_ref, degn_ref, x_ref, g_ref, disp_ref, disn_ref):
    i = pl.program_id(0)
    rel = i // (NPAD // BLK)
    dp = lax.rsqrt(degp_ref[...])
    dn = lax.rsqrt(degn_ref[...])
    disp_ref[...] = dp
    disn_ref[...] = dn
    g_ref[...] = x_ref[...] * jnp.where(rel == 0, dp, dn)


def _tc_prep(deg_p, deg_n, x):
    nb = NPAD // BLK
    return pl.pallas_call(
        _prep_body,
        grid=(2 * nb,),
        in_specs=[
            pl.BlockSpec((BLK, 1), lambda i: (i % nb, 0)),
            pl.BlockSpec((BLK, 1), lambda i: (i % nb, 0)),
            pl.BlockSpec((BLK, D), lambda i: (i % nb, 0)),
        ],
        out_specs=[
            pl.BlockSpec((BLK, D), lambda i: (i, 0)),
            pl.BlockSpec((BLK, 1), lambda i: (i % nb, 0)),
            pl.BlockSpec((BLK, 1), lambda i: (i % nb, 0)),
        ],
        out_shape=[
            jax.ShapeDtypeStruct((2 * NPAD, D), jnp.float32),
            jax.ShapeDtypeStruct((NPAD, 1), jnp.float32),
            jax.ShapeDtypeStruct((NPAD, 1), jnp.float32),
        ],
    )(deg_p, deg_n, x)


def _combine_body(sp_ref, sn_ref, h_ref, dp_ref, dn_ref,
                  wp_ref, wn_ref, bp_ref, bn_ref, hn_ref, g_ref):
    i = pl.program_id(0)
    rel = i // (NPAD // BLK)
    dp = dp_ref[...]
    dn = dn_ref[...]
    h = h_ref[...]
    up = dp * sp_ref[...] + (dp * dp) * h
    un = dn * sn_ref[...] + (dn * dn) * h
    hn = jnp.dot(up, wp_ref[...], preferred_element_type=jnp.float32)
    hn += jnp.dot(un, wn_ref[...], preferred_element_type=jnp.float32)
    hn = jnp.maximum(hn + bp_ref[...] + bn_ref[...], 0.0)
    hn_ref[...] = hn
    g_ref[...] = hn * jnp.where(rel == 0, dp, dn)


def _tc_combine(S_p, S_n, h, dis_p, dis_n, Wp, Wn, bp, bn):
    nb = NPAD // BLK
    return pl.pallas_call(
        _combine_body,
        grid=(2 * nb,),
        in_specs=[
            pl.BlockSpec((BLK, D), lambda i: (i % nb, 0)),
            pl.BlockSpec((BLK, D), lambda i: (i % nb, 0)),
            pl.BlockSpec((BLK, D), lambda i: (i % nb, 0)),
            pl.BlockSpec((BLK, 1), lambda i: (i % nb, 0)),
            pl.BlockSpec((BLK, 1), lambda i: (i % nb, 0)),
            pl.BlockSpec((D, D), lambda i: (0, 0)),
            pl.BlockSpec((D, D), lambda i: (0, 0)),
            pl.BlockSpec((1, D), lambda i: (0, 0)),
            pl.BlockSpec((1, D), lambda i: (0, 0)),
        ],
        out_specs=[
            pl.BlockSpec((BLK, D), lambda i: (i % nb, 0)),
            pl.BlockSpec((BLK, D), lambda i: (i, 0)),
        ],
        out_shape=[
            jax.ShapeDtypeStruct((N, D), jnp.float32),
            jax.ShapeDtypeStruct((2 * NPAD, D), jnp.float32),
        ],
    )(S_p, S_n, h, dis_p, dis_n, Wp, Wn, bp, bn)


def _final_body(sp_ref, sn_ref, h_ref, dp_ref, dn_ref, wp_ref, wn_ref,
                bp_ref, bn_ref, batch_ref, zn_ref, zg_ref, sums, cnts):
    i = pl.program_id(0)
    nb = NPAD // BLK
    dp = dp_ref[...]
    dn = dn_ref[...]
    h = h_ref[...]
    up = dp * sp_ref[...] + (dp * dp) * h
    un = dn * sn_ref[...] + (dn * dn) * h
    z = jnp.dot(up, wp_ref[...], preferred_element_type=jnp.float32)
    z += jnp.dot(un, wn_ref[...], preferred_element_type=jnp.float32)
    z = z + bp_ref[...] + bn_ref[...]
    zn_ref[...] = z

    b = batch_ref[...]
    valid = b < float(NG)
    zm = jnp.where(valid, z, 0.0)
    gids = lax.broadcasted_iota(jnp.int32, (1, NG), 1).astype(jnp.float32)
    oh = (b == gids).astype(jnp.float32)

    @pl.when(i == 0)
    def _():
        sums[...] = jnp.zeros_like(sums)
        cnts[...] = jnp.zeros_like(cnts)

    sums[...] += lax.dot_general(oh, zm, (((0,), (0,)), ((), ())),
                                 preferred_element_type=jnp.float32)
    cnt = jnp.sum(oh, axis=0)
    cnts[...] += jnp.broadcast_to(cnt[:, None], (NG, DL))

    @pl.when(i == nb - 1)
    def _():
        zg_ref[...] = sums[...] / jnp.maximum(cnts[...], 1.0)


def _tc_final(S_p, S_n, h, dis_p, dis_n, Wp, Wn, bp, bn, batchf):
    nb = NPAD // BLK
    return pl.pallas_call(
        _final_body,
        grid=(nb,),
        in_specs=[
            pl.BlockSpec((BLK, D), lambda i: (i, 0)),
            pl.BlockSpec((BLK, D), lambda i: (i, 0)),
            pl.BlockSpec((BLK, D), lambda i: (i, 0)),
            pl.BlockSpec((BLK, 1), lambda i: (i, 0)),
            pl.BlockSpec((BLK, 1), lambda i: (i, 0)),
            pl.BlockSpec((D, DL), lambda i: (0, 0)),
            pl.BlockSpec((D, DL), lambda i: (0, 0)),
            pl.BlockSpec((1, DL), lambda i: (0, 0)),
            pl.BlockSpec((1, DL), lambda i: (0, 0)),
            pl.BlockSpec((BLK, 1), lambda i: (i, 0)),
        ],
        out_specs=[
            pl.BlockSpec((BLK, DL), lambda i: (i, 0)),
            pl.BlockSpec((NG, DL), lambda i: (0, 0)),
        ],
        out_shape=[
            jax.ShapeDtypeStruct((N, DL), jnp.float32),
            jax.ShapeDtypeStruct((NG, DL), jnp.float32),
        ],
        scratch_shapes=[
            pltpu.VMEM((NG, DL), jnp.float32),
            pltpu.VMEM((NG, DL), jnp.float32),
        ],
    )(S_p, S_n, h, dis_p, dis_n, Wp, Wn, bp, bn, batchf)


def kernel(x, edge_index, edge_type, edge_attr, batch,
           W1p, b1p, W1n, b1n, W2p, b2p, W2n, b2n, W3p, b3p, W3n, b3n):
    src = edge_index[0]
    dst = edge_index[1]

    deg = _sc_degree(dst, edge_type, edge_attr)
    deg_p = deg[0].reshape(NPAD, 1)
    deg_n = deg[1].reshape(NPAD, 1)

    g, dis_p, dis_n = _tc_prep(deg_p, deg_n, x)

    S = _sc_spmm(g, src, dst, edge_type, edge_attr)
    h1, g = _tc_combine(S[:NPAD], S[NPAD:], x, dis_p, dis_n,
                        W1p, W1n, b1p.reshape(1, D), b1n.reshape(1, D))

    S = _sc_spmm(g, src, dst, edge_type, edge_attr)
    h2, g = _tc_combine(S[:NPAD], S[NPAD:], h1, dis_p, dis_n,
                        W2p, W2n, b2p.reshape(1, D), b2n.reshape(1, D))

    S = _sc_spmm(g, src, dst, edge_type, edge_attr)
    batchf = jnp.concatenate(
        [batch.astype(jnp.float32),
         jnp.full((NPAD - N,), 99.0, jnp.float32)]).reshape(NPAD, 1)
    z_node, z_graph = _tc_final(S[:NPAD], S[NPAD:], h2, dis_p, dis_n,
                                W3p, W3n, b3p.reshape(1, DL),
                                b3n.reshape(1, DL), batchf)
    return (z_node, z_graph)
